# baseline (speedup 1.0000x reference)
import functools

import jax
import jax.numpy as jnp
from jax import lax
from jax.experimental import pallas as pl
from jax.experimental.pallas import tpu as pltpu
from jax.experimental.pallas import tpu_sc as plsc

BB = 2
NN = 2048
S = BB * NN
D = 1024
H = 2048
E = 8
C = (2 * S) // E
ES = E * C
TRASH = ES
ROWS_PAD = ES + 8

NC = 2
NS = 16
NW = NC * NS
TPW = S // NW
CHR = 32
NCH = TPW // CHR

HB = 1024
NH = H // HB

_LANES = 128


def _route_body(x_ref, wg_ref, slot0_ref, slot1_ref, g0_ref, g1_ref):
    x = x_ref[...]
    wgp = jnp.pad(wg_ref[...], ((0, 0), (0, _LANES - E)))
    logits = lax.dot_general(
        x, wgp, (((1,), (0,)), ((), ())),
        preferred_element_type=jnp.float32)
    col = lax.broadcasted_iota(jnp.int32, (S, _LANES), 1)
    valid = col < E
    logits = jnp.where(valid, logits, jnp.float32(-1e30))
    mx = jnp.max(logits, axis=1, keepdims=True)
    ex = jnp.where(valid, jnp.exp(logits - mx), 0.0)
    probs = ex / jnp.sum(ex, axis=1, keepdims=True)
    v0 = jnp.max(probs, axis=1, keepdims=True)
    i0 = jnp.min(jnp.where(probs == v0, col, _LANES), axis=1, keepdims=True)
    m0 = col == i0
    probs1 = jnp.where(m0 | ~valid, jnp.float32(-1.0), probs)
    v1 = jnp.max(probs1, axis=1, keepdims=True)
    i1 = jnp.min(jnp.where(probs1 == v1, col, _LANES), axis=1, keepdims=True)
    m1 = col == i1
    cnt = m0.astype(jnp.int32) + (m1.astype(jnp.int32) << 16)
    inc = cnt
    d = 1
    while d < S:
        shifted = jnp.concatenate(
            [jnp.zeros((d, _LANES), jnp.int32), inc[:-d, :]], axis=0)
        inc = inc + shifted
        d *= 2
    inc0 = inc & 0xFFFF
    inc1 = inc >> 16
    cnt0 = cnt & 0xFFFF
    cnt1 = cnt >> 16
    pos0 = inc0 - cnt0
    total0 = jnp.max(inc0, axis=0, keepdims=True)
    pos1 = inc1 - cnt1 + total0
    loc0 = jnp.sum(jnp.where(m0, pos0, 0), axis=1, keepdims=True)
    loc1 = jnp.sum(jnp.where(m1, pos1, 0), axis=1, keepdims=True)
    keep0 = loc0 < C
    keep1 = loc1 < C
    slot0_ref[...] = jnp.where(keep0, i0 * C + loc0, TRASH)
    slot1_ref[...] = jnp.where(keep1, i1 * C + loc1, TRASH)
    denom = v0 + v1 + jnp.float32(1e-9)
    g0_ref[...] = v0 / denom
    g1_ref[...] = v1 / denom


def _route_call(xt, wgp):
    return pl.pallas_call(
        _route_body,
        out_shape=(
            jax.ShapeDtypeStruct((S, 1), jnp.int32),
            jax.ShapeDtypeStruct((S, 1), jnp.int32),
            jax.ShapeDtypeStruct((S, 1), jnp.float32),
            jax.ShapeDtypeStruct((S, 1), jnp.float32),
        ),
    )(xt, wgp)


def _dispatch_body(x_hbm, s0_hbm, s1_hbm, buf_hbm, idx_v, xa, xb,
                   la_s, lb_s, sa_s, sb_s):
    wid = lax.axis_index("s") * NC + lax.axis_index("c")
    base = wid * TPW
    pltpu.sync_copy(s0_hbm.at[wid], idx_v.at[0])
    pltpu.sync_copy(s1_hbm.at[wid], idx_v.at[1])
    bufs = (xa, xb)
    lsems = (la_s, lb_s)
    ssems = (sa_s, sb_s)
    lds = [None] * NCH
    scs = [None] * NCH
    lds[0] = pltpu.async_copy(x_hbm.at[pl.ds(base, CHR)], xa, la_s)
    for c in range(NCH):
        lds[c].wait()
        if c + 1 < NCH:
            if c - 1 >= 0:
                scs[c - 1][0].wait()
                scs[c - 1][1].wait()
            lds[c + 1] = pltpu.async_copy(
                x_hbm.at[pl.ds(base + (c + 1) * CHR, CHR)],
                bufs[(c + 1) % 2], lsems[(c + 1) % 2])
        src = bufs[c % 2]
        sem = ssems[c % 2]
        scs[c] = (
            pltpu.async_copy(src, buf_hbm.at[idx_v.at[0, c]], sem),
            pltpu.async_copy(src, buf_hbm.at[idx_v.at[1, c]], sem),
        )
    scs[NCH - 2][0].wait()
    scs[NCH - 2][1].wait()
    scs[NCH - 1][0].wait()
    scs[NCH - 1][1].wait()


def _dispatch_call(xt, slot0, slot1):
    mesh = plsc.VectorSubcoreMesh(core_axis_name="c", subcore_axis_name="s")
    fn = functools.partial(
        pl.kernel,
        mesh=mesh,
        out_type=jax.ShapeDtypeStruct((ROWS_PAD, D), jnp.float32),
        scratch_types=[
            pltpu.VMEM((2, NCH, CHR), jnp.int32),
            pltpu.VMEM((CHR, D), jnp.float32),
            pltpu.VMEM((CHR, D), jnp.float32),
            pltpu.SemaphoreType.DMA,
            pltpu.SemaphoreType.DMA,
            pltpu.SemaphoreType.DMA,
            pltpu.SemaphoreType.DMA,
        ],
        compiler_params=pltpu.CompilerParams(needs_layout_passes=False),
    )(_dispatch_body)
    return fn(xt, slot0, slot1)


def _ffn_body(buf_ref, w1_ref, b1_ref, w2_ref, b2_ref, out_ref):
    h_idx = pl.program_id(1)
    acc = jnp.dot(buf_ref[...], w1_ref[0],
                  preferred_element_type=jnp.float32)
    h = jnp.maximum(acc + b1_ref[0], 0.0)
    w2s = jnp.sum(w2_ref[0], axis=1, keepdims=True)
    part = jnp.dot(h, w2s, preferred_element_type=jnp.float32)

    @pl.when(h_idx == 0)
    def _():
        out_ref[...] = part + jnp.sum(b2_ref[...])

    @pl.when(h_idx != 0)
    def _():
        out_ref[...] = out_ref[...] + part


def _ffn_call(buf, W1, b1, W2, b2):
    return pl.pallas_call(
        _ffn_body,
        grid=(E, NH),
        in_specs=[
            pl.BlockSpec((C, D), lambda e, h: (e, 0)),
            pl.BlockSpec((1, D, HB), lambda e, h: (e, 0, h)),
            pl.BlockSpec((1, 1, HB), lambda e, h: (e, 0, h)),
            pl.BlockSpec((1, HB, D), lambda e, h: (e, h, 0)),
            pl.BlockSpec((1, 1, D), lambda e, h: (e, 0, 0)),
        ],
        out_specs=pl.BlockSpec((C, 1), lambda e, h: (e, 0)),
        out_shape=jax.ShapeDtypeStruct((ES, 1), jnp.float32),
    )(buf, W1, b1.reshape(E, 1, H), W2, b2.reshape(E, 1, D))


def _combine_body(esum_hbm, s0_hbm, s1_hbm, g0_hbm, g1_hbm, y_hbm,
                  es_v, sl_v, g_v, y_v):
    wid = lax.axis_index("s") * NC + lax.axis_index("c")
    pltpu.sync_copy(esum_hbm, es_v)
    pltpu.sync_copy(s0_hbm.at[wid], sl_v.at[0])
    pltpu.sync_copy(s1_hbm.at[wid], sl_v.at[1])
    pltpu.sync_copy(g0_hbm.at[wid], g_v.at[0])
    pltpu.sync_copy(g1_hbm.at[wid], g_v.at[1])
    for j in range(TPW // 16):
        sl0 = sl_v[0, pl.ds(j * 16, 16)]
        sl1 = sl_v[1, pl.ds(j * 16, 16)]
        k0 = sl0 < ES
        k1 = sl1 < ES
        gi0 = jnp.where(k0, sl0, 0)
        gi1 = jnp.where(k1, sl1, 0)
        v0 = plsc.load_gather(es_v, [gi0])
        v1 = plsc.load_gather(es_v, [gi1])
        a0 = jnp.where(k0, g_v[0, pl.ds(j * 16, 16)] * v0, 0.0)
        a1 = jnp.where(k1, g_v[1, pl.ds(j * 16, 16)] * v1, 0.0)
        y_v[pl.ds(j * 16, 16)] = a0 + a1
    pltpu.sync_copy(y_v, y_hbm.at[wid])


def _combine_call(esum, slot0, slot1, g0, g1):
    mesh = plsc.VectorSubcoreMesh(core_axis_name="c", subcore_axis_name="s")
    fn = functools.partial(
        pl.kernel,
        mesh=mesh,
        out_type=jax.ShapeDtypeStruct((NW, TPW), jnp.float32),
        scratch_types=[
            pltpu.VMEM((ES,), jnp.float32),
            pltpu.VMEM((2, TPW), jnp.int32),
            pltpu.VMEM((2, TPW), jnp.float32),
            pltpu.VMEM((TPW,), jnp.float32),
        ],
        compiler_params=pltpu.CompilerParams(needs_layout_passes=False),
    )(_combine_body)
    return fn(esum, slot0, slot1, g0, g1)


def _final_body(y_ref, o_ref):
    y = y_ref[0]
    m = jnp.max(y)
    exy = jnp.exp(y - m)
    ssum = jnp.sum(exy)
    o_ref[0] = y - (m + jnp.log(ssum))


def _final_call(y3):
    return pl.pallas_call(
        _final_body,
        grid=(BB,),
        in_specs=[pl.BlockSpec((1, NN // _LANES, _LANES),
                               lambda b: (b, 0, 0))],
        out_specs=pl.BlockSpec((1, NN // _LANES, _LANES), lambda b: (b, 0, 0)),
        out_shape=jax.ShapeDtypeStruct((BB, NN // _LANES, _LANES),
                                       jnp.float32),
    )(y3)


def kernel(x, wg, W1, b1, W2, b2):
    xt = x.reshape(S, D)
    slot0, slot1, g0, g1 = _route_call(xt, wg)
    buf = _dispatch_call(
        xt, slot0.reshape(NW, NCH, CHR), slot1.reshape(NW, NCH, CHR))
    esum = _ffn_call(buf, W1, b1, W2, b2)
    y = _combine_call(
        esum.reshape(ES), slot0.reshape(NW, TPW), slot1.reshape(NW, TPW),
        g0.reshape(NW, TPW), g1.reshape(NW, TPW))
    out = _final_call(y.reshape(BB, NN // _LANES, _LANES))
    return out.reshape(BB, NN)


# speedup vs baseline: 1.0999x; 1.0999x over previous
import functools

import jax
import jax.numpy as jnp
from jax import lax
from jax.experimental import pallas as pl
from jax.experimental.pallas import tpu as pltpu
from jax.experimental.pallas import tpu_sc as plsc

BB = 2
NN = 2048
S = BB * NN
D = 1024
H = 2048
E = 8
C = (2 * S) // E
ES = E * C
TRASH = ES
ROWS_PAD = ES + 8

NC = 2
NS = 16
NW = NC * NS
TPW = S // NW
CHR = 32
NCH = TPW // CHR

HB = 2048
NH = H // HB

_LANES = 128


def _route_body(x_ref, wg_ref, slot0_ref, slot1_ref, g0_ref, g1_ref):
    x = x_ref[...]
    wgp = wg_ref[...]
    logits = lax.dot_general(
        x, wgp, (((1,), (0,)), ((), ())),
        preferred_element_type=jnp.float32)
    col = lax.broadcasted_iota(jnp.int32, (S, _LANES), 1)
    valid = col < E
    logits = jnp.where(valid, logits, jnp.float32(-1e30))
    mx = jnp.max(logits, axis=1, keepdims=True)
    ex = jnp.where(valid, jnp.exp(logits - mx), 0.0)
    probs = ex / jnp.sum(ex, axis=1, keepdims=True)
    v0 = jnp.max(probs, axis=1, keepdims=True)
    i0 = jnp.min(jnp.where(probs == v0, col, _LANES), axis=1, keepdims=True)
    m0 = col == i0
    probs1 = jnp.where(m0 | ~valid, jnp.float32(-1.0), probs)
    v1 = jnp.max(probs1, axis=1, keepdims=True)
    i1 = jnp.min(jnp.where(probs1 == v1, col, _LANES), axis=1, keepdims=True)
    m1 = col == i1
    cnt = m0.astype(jnp.int32) + (m1.astype(jnp.int32) << 16)
    c8 = cnt[:, :E]
    nst = 16
    rs = S // nst
    cp = jnp.concatenate(
        [c8[s * rs:(s + 1) * rs, :] for s in range(nst)], axis=1)
    inc_p = cp
    d = 1
    while d < rs:
        inc_p = inc_p + jnp.concatenate(
            [jnp.zeros((d, nst * E), jnp.int32), inc_p[:-d, :]], axis=0)
        d *= 2
    stot = inc_p[rs - 1:rs, :]
    spre = stot
    gtot = stot
    for d in (E, 2 * E, 4 * E, 8 * E):
        spre = spre + jnp.concatenate(
            [jnp.zeros((1, d), jnp.int32), spre[:, :-d]], axis=1)
        gtot = gtot + jnp.concatenate(
            [gtot[:, -d:], gtot[:, :-d]], axis=1)
    pos_p = inc_p + (spre - stot)
    inc8 = jnp.concatenate(
        [pos_p[:, s * E:(s + 1) * E] for s in range(nst)], axis=0)
    pos0 = (inc8 & 0xFFFF) - (c8 & 0xFFFF)
    tot0 = (gtot & 0xFFFF)[:, :E]
    pos1 = (inc8 >> 16) - (c8 >> 16) + tot0
    m08 = m0[:, :E]
    m18 = m1[:, :E]
    loc0 = jnp.sum(jnp.where(m08, pos0, 0), axis=1, keepdims=True)
    loc1 = jnp.sum(jnp.where(m18, pos1, 0), axis=1, keepdims=True)
    keep0 = loc0 < C
    keep1 = loc1 < C
    slot0_ref[...] = jnp.where(keep0, i0 * C + loc0, TRASH)
    slot1_ref[...] = jnp.where(keep1, i1 * C + loc1, TRASH)
    denom = v0 + v1 + jnp.float32(1e-9)
    g0_ref[...] = v0 / denom
    g1_ref[...] = v1 / denom


def _route_call(xt, wgp):
    return pl.pallas_call(
        _route_body,
        out_shape=(
            jax.ShapeDtypeStruct((S, 1), jnp.int32),
            jax.ShapeDtypeStruct((S, 1), jnp.int32),
            jax.ShapeDtypeStruct((S, 1), jnp.float32),
            jax.ShapeDtypeStruct((S, 1), jnp.float32),
        ),
    )(xt, wgp)


def _dispatch_body(x_hbm, slots_hbm, buf_hbm, idx_v, xa, xb,
                   la_s, lb_s, sa_s, sb_s):
    wid = lax.axis_index("s") * NC + lax.axis_index("c")
    base = wid * TPW
    pltpu.sync_copy(slots_hbm.at[wid], idx_v)
    bufs = (xa, xb)
    lsems = (la_s, lb_s)
    ssems = (sa_s, sb_s)
    lds = [None] * NCH
    scs = [None] * NCH
    lds[0] = pltpu.async_copy(x_hbm.at[pl.ds(base, CHR)], xa, la_s)
    for c in range(NCH):
        lds[c].wait()
        if c + 1 < NCH:
            if c - 1 >= 0:
                scs[c - 1][0].wait()
                scs[c - 1][1].wait()
            lds[c + 1] = pltpu.async_copy(
                x_hbm.at[pl.ds(base + (c + 1) * CHR, CHR)],
                bufs[(c + 1) % 2], lsems[(c + 1) % 2])
        src = bufs[c % 2]
        sem = ssems[c % 2]
        scs[c] = (
            pltpu.async_copy(src, buf_hbm.at[idx_v.at[0, c]], sem),
            pltpu.async_copy(src, buf_hbm.at[idx_v.at[1, c]], sem),
        )
    scs[NCH - 2][0].wait()
    scs[NCH - 2][1].wait()
    scs[NCH - 1][0].wait()
    scs[NCH - 1][1].wait()


def _dispatch_call(xt, slots4):
    mesh = plsc.VectorSubcoreMesh(core_axis_name="c", subcore_axis_name="s")
    fn = functools.partial(
        pl.kernel,
        mesh=mesh,
        out_type=jax.ShapeDtypeStruct((ROWS_PAD, D), jnp.float32),
        scratch_types=[
            pltpu.VMEM((2, NCH, CHR), jnp.int32),
            pltpu.VMEM((CHR, D), jnp.float32),
            pltpu.VMEM((CHR, D), jnp.float32),
            pltpu.SemaphoreType.DMA,
            pltpu.SemaphoreType.DMA,
            pltpu.SemaphoreType.DMA,
            pltpu.SemaphoreType.DMA,
        ],
        compiler_params=pltpu.CompilerParams(needs_layout_passes=False),
    )(_dispatch_body)
    return fn(xt, slots4)


def _ffn_body(buf_ref, w1_ref, b1_ref, w2_ref, b2_ref, out_ref):
    h_idx = pl.program_id(1)
    acc = jnp.dot(buf_ref[...], w1_ref[0],
                  preferred_element_type=jnp.float32)
    h = jnp.maximum(acc + b1_ref[0], 0.0)
    w2s = jnp.sum(w2_ref[0], axis=1, keepdims=True)
    part = jnp.dot(h, w2s, preferred_element_type=jnp.float32)

    @pl.when(h_idx == 0)
    def _():
        out_ref[...] = part + jnp.sum(b2_ref[...])

    @pl.when(h_idx != 0)
    def _():
        out_ref[...] = out_ref[...] + part


def _ffn_call(buf, W1, b1, W2, b2):
    return pl.pallas_call(
        _ffn_body,
        grid=(E, NH),
        in_specs=[
            pl.BlockSpec((C, D), lambda e, h: (e, 0)),
            pl.BlockSpec((1, D, HB), lambda e, h: (e, 0, h)),
            pl.BlockSpec((1, 1, HB), lambda e, h: (e, 0, h)),
            pl.BlockSpec((1, HB, D), lambda e, h: (e, h, 0)),
            pl.BlockSpec((1, 1, D), lambda e, h: (e, 0, 0)),
        ],
        out_specs=pl.BlockSpec((C, 1), lambda e, h: (e, 0)),
        out_shape=jax.ShapeDtypeStruct((ES, 1), jnp.float32),
    )(buf, W1, b1.reshape(E, 1, H), W2, b2.reshape(E, 1, D))


def _combine_body(esum_hbm, slots_hbm, gates_hbm, y_hbm,
                  es_v, sl_v, g_v, y_v):
    wid = lax.axis_index("s") * NC + lax.axis_index("c")
    pltpu.sync_copy(esum_hbm, es_v)
    pltpu.sync_copy(slots_hbm.at[wid], sl_v)
    pltpu.sync_copy(gates_hbm.at[wid], g_v)
    for j in range(TPW // 16):
        sl0 = sl_v[0, pl.ds(j * 16, 16)]
        sl1 = sl_v[1, pl.ds(j * 16, 16)]
        k0 = sl0 < ES
        k1 = sl1 < ES
        gi0 = jnp.where(k0, sl0, 0)
        gi1 = jnp.where(k1, sl1, 0)
        v0 = plsc.load_gather(es_v, [gi0])
        v1 = plsc.load_gather(es_v, [gi1])
        a0 = jnp.where(k0, g_v[0, pl.ds(j * 16, 16)] * v0, 0.0)
        a1 = jnp.where(k1, g_v[1, pl.ds(j * 16, 16)] * v1, 0.0)
        y_v[pl.ds(j * 16, 16)] = a0 + a1
    pltpu.sync_copy(y_v, y_hbm.at[wid])


def _combine_call(esum, slots2, gates2):
    mesh = plsc.VectorSubcoreMesh(core_axis_name="c", subcore_axis_name="s")
    fn = functools.partial(
        pl.kernel,
        mesh=mesh,
        out_type=jax.ShapeDtypeStruct((NW, TPW), jnp.float32),
        scratch_types=[
            pltpu.VMEM((ES,), jnp.float32),
            pltpu.VMEM((2, TPW), jnp.int32),
            pltpu.VMEM((2, TPW), jnp.float32),
            pltpu.VMEM((TPW,), jnp.float32),
        ],
        compiler_params=pltpu.CompilerParams(needs_layout_passes=False),
    )(_combine_body)
    return fn(esum, slots2, gates2)


def _final_body(y_ref, o_ref):
    y = y_ref[0]
    m = jnp.max(y)
    exy = jnp.exp(y - m)
    ssum = jnp.sum(exy)
    o_ref[0] = y - (m + jnp.log(ssum))


def _final_call(y3):
    return pl.pallas_call(
        _final_body,
        grid=(BB,),
        in_specs=[pl.BlockSpec((1, NN // _LANES, _LANES),
                               lambda b: (b, 0, 0))],
        out_specs=pl.BlockSpec((1, NN // _LANES, _LANES), lambda b: (b, 0, 0)),
        out_shape=jax.ShapeDtypeStruct((BB, NN // _LANES, _LANES),
                                       jnp.float32),
    )(y3)


def kernel(x, wg, W1, b1, W2, b2):
    xt = x.reshape(S, D)
    wgp = jnp.pad(wg, ((0, 0), (0, _LANES - E)))
    slot0, slot1, g0, g1 = _route_call(xt, wgp)
    slots2 = jnp.stack(
        [slot0.reshape(NW, TPW), slot1.reshape(NW, TPW)], axis=1)
    slots4 = slots2.reshape(NW, 2, NCH, CHR)
    buf = _dispatch_call(xt, slots4)
    esum = _ffn_call(buf, W1, b1, W2, b2)
    gates2 = jnp.stack([g0.reshape(NW, TPW), g1.reshape(NW, TPW)], axis=1)
    y = _combine_call(esum.reshape(ES), slots2, gates2)
    out = _final_call(y.reshape(BB, NN // _LANES, _LANES))
    return out.reshape(BB, NN)
